# baseline (speedup 1.0000x reference)
import functools

import jax
import jax.numpy as jnp
from jax import lax
from jax.experimental import pallas as pl
from jax.experimental.pallas import tpu as pltpu
from jax.experimental.pallas import tpu_sc as plsc

N = 10000
NP = 10240
E = 320000
D = 128
NC = 2
NS = 16
EPT = E // NS
CH = 64
NCHUNK = 320
SDC = NCHUNK + 4
G = 8
RPT = NP // NS

@functools.cache
def _get_sc_degrees():
    mesh = plsc.VectorSubcoreMesh(core_axis_name="c", subcore_axis_name="s")
    return functools.partial(
        pl.kernel,
        mesh=mesh,
        out_type=(
            jax.ShapeDtypeStruct((NC * NS, NP), jnp.float32),
            jax.ShapeDtypeStruct((NC * NS, NP), jnp.float32),
        ),
        compiler_params=pltpu.CompilerParams(needs_layout_passes=False),
        scratch_types=[
            pltpu.VMEM((G, 2, CH), jnp.int32),
            pltpu.VMEM((NP,), jnp.float32),
            pltpu.VMEM((NP,), jnp.float32),
        ],
    )(_sc_degrees_body)


def _sc_degrees_body(sd_hbm, osrc_hbm, odst_hbm, stg, acc_s, acc_d):
    cid = lax.axis_index("c")
    sid = lax.axis_index("s")
    wid = cid * NS + sid
    off = cid * NP
    zero16 = jnp.zeros((16,), jnp.float32)
    ones16 = jnp.ones((16,), jnp.float32)

    def zstep(j, carry):
        acc_s[pl.ds(j * 16, 16)] = zero16
        acc_d[pl.ds(j * 16, 16)] = zero16
        return carry

    lax.fori_loop(0, NP // 16, zstep, 0)

    def step(t, carry):
        pltpu.sync_copy(sd_hbm.at[wid, pl.ds(t * G, G)], stg)
        for g in range(G):
            for kk in range(CH // 16):
                plsc.addupdate_scatter(
                    acc_s, [stg[g, 0, pl.ds(kk * 16, 16)] - off], ones16)
                plsc.addupdate_scatter(
                    acc_d, [stg[g, 1, pl.ds(kk * 16, 16)]], ones16)
        return carry

    lax.fori_loop(0, NCHUNK // G, step, 0)
    pltpu.sync_copy(acc_s, osrc_hbm.at[wid])
    pltpu.sync_copy(acc_d, odst_hbm.at[wid])


@functools.cache
def _get_sc_spmm():
    mesh = plsc.VectorSubcoreMesh(core_axis_name="c", subcore_axis_name="s")
    return functools.partial(
        pl.kernel,
        mesh=mesh,
        out_type=jax.ShapeDtypeStruct((NC * NP, D), jnp.float32),
        scratch_types=[
            pltpu.VMEM((8, 2, CH), jnp.int32),
            pltpu.VMEM((4, CH, D), jnp.float32),
            pltpu.VMEM_SHARED((NP, D), jnp.float32),
        ] + [pltpu.SemaphoreType.DMA] * 16,
    )(_sc_spmm_body)


def _sc_spmm_body(z_hbm, sd_hbm, zeros_hbm, out_hbm, idxb, rows, acc,
                  *sems):
    cid = lax.axis_index("c")
    sid = lax.axis_index("s")
    wid = cid * NS + sid
    isems = sems[0:8]
    gsems = sems[8:12]
    ssems = sems[12:16]
    pltpu.sync_copy(zeros_hbm, acc.at[pl.ds(sid * RPT, RPT)])
    plsc.subcore_barrier()

    for k in range(4):
        pltpu.async_copy(sd_hbm.at[wid, k], idxb.at[k], isems[k])

    def drain_scatter(br, si):
        pltpu.make_async_copy(rows.at[br], acc.at[idxb.at[si, 1]],
                              ssems[br]).wait()

    def step8(j, carry):
        i0 = 8 * j
        for k in range(8):
            br = k % 4
            sp = (k + 4) % 8
            if k < 4:
                @pl.when(j > 0)
                def _():
                    drain_scatter(br, sp)
            else:
                drain_scatter(br, sp)
            pltpu.async_copy(sd_hbm.at[wid, i0 + k + 4], idxb.at[sp],
                             isems[sp])
            pltpu.make_async_copy(sd_hbm.at[wid, 0], idxb.at[k],
                                  isems[k]).wait()
            pltpu.async_copy(z_hbm.at[idxb.at[k, 0]], rows.at[br],
                             gsems[br]).wait()
            pltpu.async_copy(rows.at[br], acc.at[idxb.at[k, 1]], ssems[br],
                             add=True)
        return carry

    lax.fori_loop(0, NCHUNK // 8, step8, 0)
    for k in range(4):
        drain_scatter(k, (k + 4) % 8)
        pltpu.make_async_copy(sd_hbm.at[wid, 0], idxb.at[k], isems[k]).wait()
    plsc.subcore_barrier()
    pltpu.sync_copy(acc.at[pl.ds(sid * RPT, RPT)],
                    out_hbm.at[pl.ds(cid * NP + sid * RPT, RPT)])


def _tc_prep_body(hs_ref, hd_ref, xu_ref, xi_ref, w1c_ref, w1b_ref,
                  z_ref, s4_ref):
    hs = hs_ref[...]
    hd = hd_ref[...]
    deg_sc = jnp.sum(hs[:, 0:NS], axis=1, keepdims=True)
    deg_sb = jnp.sum(hs[:, NS:2 * NS], axis=1, keepdims=True)
    deg_dc = jnp.sum(hd[:, 0:NS], axis=1, keepdims=True)
    deg_db = jnp.sum(hd[:, NS:2 * NS], axis=1, keepdims=True)
    s4 = lax.rsqrt(jnp.maximum(
        jnp.concatenate([deg_sc, deg_sb, deg_dc, deg_db], axis=1), 1.0))
    s4_ref[...] = s4
    s_out_c = s4[0:N, 0:1]
    s_out_b = s4[0:N, 1:2]
    z_ref[0:N, :] = jnp.dot(xu_ref[...] * s_out_c, w1c_ref[...],
                            preferred_element_type=jnp.float32)
    z_ref[NP:NP + N, :] = jnp.dot(xi_ref[...] * s_out_b, w1b_ref[...],
                                  preferred_element_type=jnp.float32)
    z_ref[N:NP, :] = jnp.zeros((NP - N, D), jnp.float32)
    z_ref[NP + N:, :] = jnp.zeros((NP - N, D), jnp.float32)


def _tc_mid_body(p_ref, s4_ref, b1c_ref, b1b_ref, w2c_ref, w2b_ref, z_ref):
    s4 = s4_ref[...]
    s_out_c, s_out_b = s4[0:N, 0:1], s4[0:N, 1:2]
    s_in_c, s_in_b = s4[0:N, 2:3], s4[0:N, 3:4]
    h_item = jnp.maximum(p_ref[0:N, :] * s_in_c + b1c_ref[...], 0.0)
    h_user = jnp.maximum(p_ref[NP:NP + N, :] * s_in_b + b1b_ref[...], 0.0)
    z_ref[0:N, :] = jnp.dot(h_user * s_out_c, w2c_ref[...],
                            preferred_element_type=jnp.float32)
    z_ref[NP:NP + N, :] = jnp.dot(h_item * s_out_b, w2b_ref[...],
                                  preferred_element_type=jnp.float32)
    z_ref[N:NP, :] = jnp.zeros((NP - N, D), jnp.float32)
    z_ref[NP + N:, :] = jnp.zeros((NP - N, D), jnp.float32)


def _tc_final_body(q_ref, s4_ref, b2c_ref, b2b_ref, ou_ref, oi_ref):
    s4 = s4_ref[...]
    s_in_c, s_in_b = s4[0:N, 2:3], s4[0:N, 3:4]
    oi_ref[...] = q_ref[0:N, :] * s_in_c + b2c_ref[...]
    ou_ref[...] = q_ref[NP:NP + N, :] * s_in_b + b2b_ref[...]


_tc_prep = pl.pallas_call(
    _tc_prep_body,
    out_shape=(jax.ShapeDtypeStruct((NC * NP, D), jnp.float32),
               jax.ShapeDtypeStruct((NP, 4), jnp.float32)))
_tc_mid = pl.pallas_call(
    _tc_mid_body,
    out_shape=jax.ShapeDtypeStruct((NC * NP, D), jnp.float32))
_tc_final = pl.pallas_call(
    _tc_final_body,
    out_shape=(jax.ShapeDtypeStruct((N, D), jnp.float32),
               jax.ShapeDtypeStruct((N, D), jnp.float32)))


def kernel(x_user, x_item, edge_index_clicks, edge_index_clicked_by,
           W1_clicks, b1_clicks, W1_clicked_by, b1_clicked_by,
           W2_clicks, b2_clicks, W2_clicked_by, b2_clicked_by):
    src_c = edge_index_clicks[0].astype(jnp.int32)
    dst_c = edge_index_clicks[1].astype(jnp.int32)
    src_b = edge_index_clicked_by[0].astype(jnp.int32)
    dst_b = edge_index_clicked_by[1].astype(jnp.int32)

    pad = SDC * CH - EPT

    def _pack(src, dst, src_pad):
        s2 = jnp.concatenate(
            [src.reshape(NS, EPT),
             jnp.full((NS, pad), src_pad, jnp.int32)], axis=1)
        d2 = jnp.concatenate(
            [dst.reshape(NS, EPT),
             jnp.full((NS, pad), N, jnp.int32)], axis=1)
        return jnp.stack([s2.reshape(NS, SDC, CH),
                          d2.reshape(NS, SDC, CH)], axis=2)

    sd = jnp.concatenate([_pack(src_c, dst_c, N),
                          _pack(src_b + NP, dst_b, NP + N)], axis=0)

    zeros_d = jnp.zeros((RPT, D), jnp.float32)

    hsrc, hdst = _get_sc_degrees()(sd)
    hsrcT = hsrc.T
    hdstT = hdst.T

    b1c = b1_clicks.reshape(1, D)
    b1b = b1_clicked_by.reshape(1, D)
    b2c = b2_clicks.reshape(1, D)
    b2b = b2_clicked_by.reshape(1, D)

    spmm = _get_sc_spmm()
    z1, s4 = _tc_prep(hsrcT, hdstT, x_user, x_item, W1_clicks, W1_clicked_by)
    p1 = spmm(z1, sd, zeros_d)
    z2 = _tc_mid(p1, s4, b1c, b1b, W2_clicks, W2_clicked_by)
    p2 = spmm(z2, sd, zeros_d)
    out_user, out_item = _tc_final(p2, s4, b2c, b2b)
    return (out_user, out_item)


# speedup vs baseline: 1.2098x; 1.2098x over previous
import functools

import jax
import jax.numpy as jnp
from jax import lax
from jax.experimental import pallas as pl
from jax.experimental.pallas import tpu as pltpu
from jax.experimental.pallas import tpu_sc as plsc

N = 10000
NP = 10240
E = 320000
D = 128
NC = 2
NS = 16
EPT = E // NS
CH = 128
NCHUNK = 160
SDC = NCHUNK + 2
G = 8
RPT = NP // NS

@functools.cache
def _get_sc_degrees():
    mesh = plsc.VectorSubcoreMesh(core_axis_name="c", subcore_axis_name="s")
    return functools.partial(
        pl.kernel,
        mesh=mesh,
        out_type=(
            jax.ShapeDtypeStruct((NC * NS, NP), jnp.float32),
            jax.ShapeDtypeStruct((NC * NS, NP), jnp.float32),
        ),
        compiler_params=pltpu.CompilerParams(needs_layout_passes=False),
        scratch_types=[
            pltpu.VMEM((G, 2, CH), jnp.int32),
            pltpu.VMEM((NP,), jnp.float32),
            pltpu.VMEM((NP,), jnp.float32),
        ],
    )(_sc_degrees_body)


def _sc_degrees_body(sd_hbm, osrc_hbm, odst_hbm, stg, acc_s, acc_d):
    cid = lax.axis_index("c")
    sid = lax.axis_index("s")
    wid = cid * NS + sid
    off = cid * NP
    zero16 = jnp.zeros((16,), jnp.float32)
    ones16 = jnp.ones((16,), jnp.float32)

    def zstep(j, carry):
        acc_s[pl.ds(j * 16, 16)] = zero16
        acc_d[pl.ds(j * 16, 16)] = zero16
        return carry

    lax.fori_loop(0, NP // 16, zstep, 0)

    def step(t, carry):
        pltpu.sync_copy(sd_hbm.at[wid, pl.ds(t * G, G)], stg)
        for g in range(G):
            for kk in range(CH // 16):
                plsc.addupdate_scatter(
                    acc_s, [stg[g, 0, pl.ds(kk * 16, 16)] - off], ones16)
                plsc.addupdate_scatter(
                    acc_d, [stg[g, 1, pl.ds(kk * 16, 16)]], ones16)
        return carry

    lax.fori_loop(0, NCHUNK // G, step, 0)
    pltpu.sync_copy(acc_s, osrc_hbm.at[wid])
    pltpu.sync_copy(acc_d, odst_hbm.at[wid])


@functools.cache
def _get_sc_spmm():
    mesh = plsc.VectorSubcoreMesh(core_axis_name="c", subcore_axis_name="s")
    return functools.partial(
        pl.kernel,
        mesh=mesh,
        out_type=jax.ShapeDtypeStruct((NC * NP, D), jnp.float32),
        scratch_types=[
            pltpu.VMEM((4, 2, CH), jnp.int32),
            pltpu.VMEM((2, CH, D), jnp.float32),
            pltpu.VMEM_SHARED((NP, D), jnp.float32),
            pltpu.SemaphoreType.DMA,
            pltpu.SemaphoreType.DMA,
            pltpu.SemaphoreType.DMA,
            pltpu.SemaphoreType.DMA,
            pltpu.SemaphoreType.DMA,
            pltpu.SemaphoreType.DMA,
            pltpu.SemaphoreType.DMA,
            pltpu.SemaphoreType.DMA,
        ],
    )(_sc_spmm_body)


def _sc_spmm_body(z_hbm, sd_hbm, zeros_hbm, out_hbm, idxb, rows, acc,
                  isem0, isem1, isem2, isem3, gsem0, gsem1, ssem0, ssem1):
    cid = lax.axis_index("c")
    sid = lax.axis_index("s")
    wid = cid * NS + sid
    isems = (isem0, isem1, isem2, isem3)
    gsems = (gsem0, gsem1)
    ssems = (ssem0, ssem1)
    pltpu.sync_copy(zeros_hbm, acc.at[pl.ds(sid * RPT, RPT)])
    plsc.subcore_barrier()

    pltpu.async_copy(sd_hbm.at[wid, 0], idxb.at[0], isem0)
    pltpu.async_copy(sd_hbm.at[wid, 1], idxb.at[1], isem1)

    def drain_scatter(br, si):
        pltpu.make_async_copy(rows.at[br], acc.at[idxb.at[si, 1]],
                              ssems[br]).wait()

    def step4(j, carry):
        i0 = 4 * j
        for k in range(4):
            br, si = k % 2, k
            sp = (k + 2) % 4
            if k < 2:
                @pl.when(j > 0)
                def _():
                    drain_scatter(br, sp)
            else:
                drain_scatter(br, sp)
            pltpu.async_copy(sd_hbm.at[wid, i0 + k + 2], idxb.at[sp],
                             isems[sp])
            pltpu.make_async_copy(sd_hbm.at[wid, 0], idxb.at[si],
                                  isems[si]).wait()
            pltpu.async_copy(z_hbm.at[idxb.at[si, 0]], rows.at[br],
                             gsems[br]).wait()
            pltpu.async_copy(rows.at[br], acc.at[idxb.at[si, 1]], ssems[br],
                             add=True)
        return carry

    lax.fori_loop(0, NCHUNK // 4, step4, 0)
    drain_scatter(0, 2)
    drain_scatter(1, 3)
    pltpu.make_async_copy(sd_hbm.at[wid, 0], idxb.at[0], isem0).wait()
    pltpu.make_async_copy(sd_hbm.at[wid, 0], idxb.at[1], isem1).wait()
    plsc.subcore_barrier()
    pltpu.sync_copy(acc.at[pl.ds(sid * RPT, RPT)],
                    out_hbm.at[pl.ds(cid * NP + sid * RPT, RPT)])


def _tc_prep_body(hs_ref, hd_ref, xu_ref, xi_ref, w1c_ref, w1b_ref,
                  z_ref, s4_ref):
    hs = hs_ref[...]
    hd = hd_ref[...]
    deg_sc = jnp.sum(hs[:, 0:NS], axis=1, keepdims=True)
    deg_sb = jnp.sum(hs[:, NS:2 * NS], axis=1, keepdims=True)
    deg_dc = jnp.sum(hd[:, 0:NS], axis=1, keepdims=True)
    deg_db = jnp.sum(hd[:, NS:2 * NS], axis=1, keepdims=True)
    s4 = lax.rsqrt(jnp.maximum(
        jnp.concatenate([deg_sc, deg_sb, deg_dc, deg_db], axis=1), 1.0))
    s4_ref[...] = s4
    s_out_c = s4[0:N, 0:1]
    s_out_b = s4[0:N, 1:2]
    z_ref[0:N, :] = jnp.dot(xu_ref[...] * s_out_c, w1c_ref[...],
                            preferred_element_type=jnp.float32)
    z_ref[NP:NP + N, :] = jnp.dot(xi_ref[...] * s_out_b, w1b_ref[...],
                                  preferred_element_type=jnp.float32)
    z_ref[N:NP, :] = jnp.zeros((NP - N, D), jnp.float32)
    z_ref[NP + N:, :] = jnp.zeros((NP - N, D), jnp.float32)


def _tc_mid_body(p_ref, s4_ref, b1c_ref, b1b_ref, w2c_ref, w2b_ref, z_ref):
    s4 = s4_ref[...]
    s_out_c, s_out_b = s4[0:N, 0:1], s4[0:N, 1:2]
    s_in_c, s_in_b = s4[0:N, 2:3], s4[0:N, 3:4]
    h_item = jnp.maximum(p_ref[0:N, :] * s_in_c + b1c_ref[...], 0.0)
    h_user = jnp.maximum(p_ref[NP:NP + N, :] * s_in_b + b1b_ref[...], 0.0)
    z_ref[0:N, :] = jnp.dot(h_user * s_out_c, w2c_ref[...],
                            preferred_element_type=jnp.float32)
    z_ref[NP:NP + N, :] = jnp.dot(h_item * s_out_b, w2b_ref[...],
                                  preferred_element_type=jnp.float32)
    z_ref[N:NP, :] = jnp.zeros((NP - N, D), jnp.float32)
    z_ref[NP + N:, :] = jnp.zeros((NP - N, D), jnp.float32)


def _tc_final_body(q_ref, s4_ref, b2c_ref, b2b_ref, ou_ref, oi_ref):
    s4 = s4_ref[...]
    s_in_c, s_in_b = s4[0:N, 2:3], s4[0:N, 3:4]
    oi_ref[...] = q_ref[0:N, :] * s_in_c + b2c_ref[...]
    ou_ref[...] = q_ref[NP:NP + N, :] * s_in_b + b2b_ref[...]


_tc_prep = pl.pallas_call(
    _tc_prep_body,
    out_shape=(jax.ShapeDtypeStruct((NC * NP, D), jnp.float32),
               jax.ShapeDtypeStruct((NP, 4), jnp.float32)))
_tc_mid = pl.pallas_call(
    _tc_mid_body,
    out_shape=jax.ShapeDtypeStruct((NC * NP, D), jnp.float32))
_tc_final = pl.pallas_call(
    _tc_final_body,
    out_shape=(jax.ShapeDtypeStruct((N, D), jnp.float32),
               jax.ShapeDtypeStruct((N, D), jnp.float32)))


def kernel(x_user, x_item, edge_index_clicks, edge_index_clicked_by,
           W1_clicks, b1_clicks, W1_clicked_by, b1_clicked_by,
           W2_clicks, b2_clicks, W2_clicked_by, b2_clicked_by):
    src_c = edge_index_clicks[0].astype(jnp.int32)
    dst_c = edge_index_clicks[1].astype(jnp.int32)
    src_b = edge_index_clicked_by[0].astype(jnp.int32)
    dst_b = edge_index_clicked_by[1].astype(jnp.int32)

    pad = SDC * CH - EPT

    def _pack(src, dst, src_pad):
        s2 = jnp.concatenate(
            [src.reshape(NS, EPT),
             jnp.full((NS, pad), src_pad, jnp.int32)], axis=1)
        d2 = jnp.concatenate(
            [dst.reshape(NS, EPT),
             jnp.full((NS, pad), N, jnp.int32)], axis=1)
        return jnp.stack([s2.reshape(NS, SDC, CH),
                          d2.reshape(NS, SDC, CH)], axis=2)

    sd = jnp.concatenate([_pack(src_c, dst_c, N),
                          _pack(src_b + NP, dst_b, NP + N)], axis=0)

    zeros_d = jnp.zeros((RPT, D), jnp.float32)

    hsrc, hdst = _get_sc_degrees()(sd)
    hsrcT = hsrc.T
    hdstT = hdst.T

    b1c = b1_clicks.reshape(1, D)
    b1b = b1_clicked_by.reshape(1, D)
    b2c = b2_clicks.reshape(1, D)
    b2b = b2_clicked_by.reshape(1, D)

    spmm = _get_sc_spmm()
    z1, s4 = _tc_prep(hsrcT, hdstT, x_user, x_item, W1_clicks, W1_clicked_by)
    p1 = spmm(z1, sd, zeros_d)
    z2 = _tc_mid(p1, s4, b1c, b1b, W2_clicks, W2_clicked_by)
    p2 = spmm(z2, sd, zeros_d)
    out_user, out_item = _tc_final(p2, s4, b2c, b2b)
    return (out_user, out_item)
